# baseline (speedup 1.0000x reference)
import functools

import jax
import jax.numpy as jnp
from jax import lax
from jax.experimental import pallas as pl
from jax.experimental.pallas import tpu as pltpu
from jax.experimental.pallas import tpu_sc as plsc

VOCAB = 1_000_000
DIM = 64
BATCH = 16384
NUM_CORES = 2
NUM_SUBCORES = 16
NUM_WORKERS = NUM_CORES * NUM_SUBCORES
B_PER_W = BATCH // NUM_WORKERS
CHUNK = 128
NCHUNK = B_PER_W // CHUNK


@functools.partial(
    pl.kernel,
    mesh=plsc.VectorSubcoreMesh(core_axis_name="c", subcore_axis_name="s"),
    out_type=jax.ShapeDtypeStruct((BATCH, DIM), jnp.float32),
    scratch_types=[
        pltpu.VMEM((NCHUNK, CHUNK), jnp.int32),
        pltpu.VMEM((NCHUNK, CHUNK, DIM), jnp.float32),
        pltpu.SemaphoreType.DMA,
    ],
    compiler_params=pltpu.CompilerParams(use_tc_tiling_on_sc=False),
)
def _sc_gather(idx_hbm, table_hbm, out_hbm, idx_v, rows_v, sem):
    wid = lax.axis_index("s") * NUM_CORES + lax.axis_index("c")
    base = wid * B_PER_W
    for j in range(NCHUNK):
        pltpu.sync_copy(idx_hbm.at[pl.ds(base + j * CHUNK, CHUNK)], idx_v.at[j])
    copies = [
        pltpu.async_copy(table_hbm.at[idx_v.at[j]], rows_v.at[j], sem)
        for j in range(NCHUNK)
    ]
    for j in range(NCHUNK):
        copies[j].wait()
        pltpu.sync_copy(rows_v.at[j], out_hbm.at[pl.ds(base + j * CHUNK, CHUNK)])


def kernel(target_word, target_embedding):
    return _sc_gather(target_word.astype(jnp.int32), target_embedding)


# speedup vs baseline: 2.9420x; 2.9420x over previous
import functools

import jax
import jax.numpy as jnp
from jax import lax
from jax.experimental import pallas as pl
from jax.experimental.pallas import tpu as pltpu
from jax.experimental.pallas import tpu_sc as plsc

VOCAB = 1_000_000
DIM = 64
BATCH = 16384
NUM_CORES = 2
NUM_SUBCORES = 16
NUM_WORKERS = NUM_CORES * NUM_SUBCORES
B_PER_W = BATCH // NUM_WORKERS
NBUF = 8
LANE = 128
NROUND = B_PER_W // NBUF


@functools.partial(
    pl.kernel,
    mesh=plsc.VectorSubcoreMesh(core_axis_name="c", subcore_axis_name="s"),
    out_type=jax.ShapeDtypeStruct((BATCH * DIM,), jnp.float32),
    scratch_types=[
        pltpu.VMEM((B_PER_W,), jnp.int32),
        pltpu.VMEM((NBUF * DIM, LANE), jnp.float32),
        pltpu.VMEM((NBUF * DIM,), jnp.float32),
        pltpu.SemaphoreType.DMA,
        pltpu.SemaphoreType.DMA,
    ],
    compiler_params=pltpu.CompilerParams(
        disable_bounds_checks=True, needs_layout_passes=False
    ),
)
def _sc_gather(idx_hbm, tt_hbm, out_hbm, idx_v, blk_v, row_v, sem, osem):
    wid = lax.axis_index("s") * NUM_CORES + lax.axis_index("c")
    base = wid * B_PER_W
    pltpu.sync_copy(idx_hbm.at[pl.ds(base, B_PER_W)], idx_v)
    lane16 = lax.iota(jnp.int32, 16)

    def widx(j):
        grp16 = pl.multiple_of((j // 16) * 16, 8)
        wv = idx_v[pl.ds(grp16, 16)]
        return jnp.max(wv * jnp.where(lane16 == j % 16, 1, 0))

    def fire(j, slot):
        cb = pl.multiple_of((widx(j) // LANE) * LANE, LANE)
        pltpu.async_copy(
            tt_hbm.at[:, pl.ds(cb, LANE)],
            blk_v.at[pl.ds(slot * DIM, DIM)],
            sem,
        )

    for b in range(NBUF):
        fire(b, b)

    def round_body(g, _):
        for b in range(NBUF):
            j = g * NBUF + b
            pltpu.make_async_copy(
                tt_hbm.at[:, pl.ds(0, LANE)],
                blk_v.at[pl.ds(b * DIM, DIM)],
                sem,
            ).wait()
            @pl.when(g > 0)
            def _():
                pltpu.make_async_copy(
                    row_v.at[pl.ds(b * DIM, DIM)],
                    out_hbm.at[pl.ds(0, DIM)],
                    osem,
                ).wait()
            col = widx(j) % LANE
            col16 = jnp.full((16,), 0, jnp.int32) + col
            for k in range(DIM // 16):
                row_v[pl.ds(b * DIM + k * 16, 16)] = plsc.load_gather(
                    blk_v, [b * DIM + k * 16 + lane16, col16]
                )
            pltpu.async_copy(
                row_v.at[pl.ds(b * DIM, DIM)],
                out_hbm.at[pl.ds((base + j) * DIM, DIM)],
                osem,
            )
            @pl.when(j + NBUF < B_PER_W)
            def _():
                fire(j + NBUF, b)
        return _

    lax.fori_loop(0, NROUND, round_body, None)
    for b in range(NBUF):
        pltpu.make_async_copy(
            row_v.at[pl.ds(b * DIM, DIM)],
            out_hbm.at[pl.ds(0, DIM)],
            osem,
        ).wait()


def kernel(target_word, target_embedding):
    flat = _sc_gather(target_word.astype(jnp.int32), target_embedding.T)
    return flat.reshape(BATCH, DIM)
